# baseline (speedup 1.0000x reference)
import jax
import jax.numpy as jnp
from jax import lax
from jax.experimental import pallas as pl
from jax.experimental.pallas import tpu as pltpu
from jax.experimental.pallas import tpu_sc as plsc

NC = 2
NS = 16
C = 128
NP = 2


def _seg_sum_split(table, srcs, dsts, half):
  _, n, dh = table.shape
  _, kc, _ = srcs.shape
  assert kc % 2 == 0 and n % NS == 0 and half % (NS * 8) == 0
  nps = n // NS
  rpw = half // NS
  ZC = 128
  acc_rows = half + ZC
  zch = acc_rows // ZC
  zrounds = -(-zch // NS)

  mesh = plsc.VectorSubcoreMesh(core_axis_name="c", subcore_axis_name="s")

  def body(tbl_hbm, src_hbm, dst_hbm, zero_hbm, out_hbm,
           src_v, dst_v, bufs, zrows_v, acc, tbl_spm, gsems, ssems, tsem):
    c = lax.axis_index("c")
    s = lax.axis_index("s")

    pltpu.async_copy(tbl_hbm.at[c, pl.ds(s * nps, nps)],
                     tbl_spm.at[pl.ds(s * nps, nps)], tsem)
    pltpu.sync_copy(zero_hbm, zrows_v)
    pltpu.sync_copy(src_hbm.at[s], src_v)
    pltpu.make_async_copy(tbl_hbm.at[c, pl.ds(s * nps, nps)],
                          tbl_spm.at[pl.ds(s * nps, nps)], tsem).wait()

    def gather(j, b):
      pltpu.async_copy(tbl_spm.at[src_v.at[j]], bufs[b], gsems[b])

    def gather_wait(j, b):
      pltpu.make_async_copy(tbl_spm.at[src_v.at[j]], bufs[b], gsems[b]).wait()

    def scatter(j, b):
      pltpu.async_copy(bufs[b], acc.at[dst_v.at[j]], ssems[b], add=True)

    def scatter_wait(j, b):
      pltpu.make_async_copy(bufs[b], acc.at[dst_v.at[j]], ssems[b]).wait()

    for p in range(NP):
      for t in range(zrounds):
        @pl.when(s + NS * t < zch)
        def _(t=t):
          pltpu.sync_copy(zrows_v, acc.at[pl.ds((s + NS * t) * ZC, ZC)])
      pltpu.sync_copy(dst_hbm.at[p, s], dst_v)
      plsc.subcore_barrier()

      gather(0, 0)

      def pair(i, carry):
        j = 2 * i
        @pl.when(i > 0)
        def _():
          scatter_wait(j - 1, 1)
        gather(j + 1, 1)
        gather_wait(j, 0)
        scatter(j, 0)
        scatter_wait(j, 0)
        @pl.when(i + 1 < kc // 2)
        def _():
          gather(j + 2, 0)
        gather_wait(j + 1, 1)
        scatter(j + 1, 1)
        return carry

      lax.fori_loop(0, kc // 2, pair, 0)
      scatter_wait(kc - 1, 1)
      plsc.subcore_barrier()

      pltpu.sync_copy(acc.at[pl.ds(s * rpw, rpw)],
                      out_hbm.at[c, pl.ds(p * half + s * rpw, rpw)])
      if p + 1 < NP:
        plsc.subcore_barrier()

  zeros = jnp.zeros((ZC, dh), jnp.float32)
  return pl.kernel(
      body,
      out_type=jax.ShapeDtypeStruct((NC, NP * half, dh), jnp.float32),
      mesh=mesh,
      compiler_params=pltpu.CompilerParams(use_tc_tiling_on_sc=False),
      scratch_types=[
          pltpu.VMEM((kc, C), jnp.int32),
          pltpu.VMEM((kc, C), jnp.int32),
          [pltpu.VMEM((C, dh), jnp.float32)] * 2,
          pltpu.VMEM((ZC, dh), jnp.float32),
          pltpu.VMEM_SHARED((acc_rows, dh), jnp.float32),
          pltpu.VMEM_SHARED((n, dh), jnp.float32),
          [pltpu.SemaphoreType.DMA] * 2,
          [pltpu.SemaphoreType.DMA] * 2,
          pltpu.SemaphoreType.DMA,
      ],
  )(table, srcs, dsts, zeros)


def _linear(p, w, b, relu, split_out, n):
  _, rows, dh = p.shape
  dout = w.shape[1]
  blk = 2000
  assert n % blk == 0

  def body(p_ref, w_ref, b_ref, o_ref):
    ssum = jnp.concatenate([p_ref[0], p_ref[1]], axis=1)
    y = lax.dot_general(ssum, w_ref[...], (((1,), (0,)), ((), ())),
                        preferred_element_type=jnp.float32,
                        precision=lax.Precision.HIGHEST)
    y = y + b_ref[...]
    if relu:
      y = jnp.maximum(y, 0.0)
    if split_out:
      o_ref[0] = y[:, :dout // 2]
      o_ref[1] = y[:, dout // 2:]
    else:
      o_ref[...] = y

  if split_out:
    out_shape = jax.ShapeDtypeStruct((2, n, dout // 2), jnp.float32)
    out_specs = pl.BlockSpec((2, blk, dout // 2), lambda i: (0, i, 0))
  else:
    out_shape = jax.ShapeDtypeStruct((n, dout), jnp.float32)
    out_specs = pl.BlockSpec((blk, dout), lambda i: (i, 0))

  return pl.pallas_call(
      body,
      grid=(n // blk,),
      in_specs=[
          pl.BlockSpec((2, blk, dh), lambda i: (0, i, 0)),
          pl.BlockSpec((dh * 2, dout), lambda i: (0, 0)),
          pl.BlockSpec((1, dout), lambda i: (0, 0)),
      ],
      out_specs=out_specs,
      out_shape=out_shape,
  )(p, w, b.reshape(1, dout))


def kernel(x, edge_index, W1, b1, W2, b2):
  n, d = x.shape
  dh = d // 2
  e = edge_index.shape[1]
  src = edge_index[0].astype(jnp.int32)
  dst = edge_index[1].astype(jnp.int32)

  kc = 2 * (-(-e // (NS * C * 2)))
  e_pad = kc * NS * C
  half = -(-n // (2 * 128)) * 128

  pad = e_pad - e
  src_p = jnp.concatenate([src, jnp.zeros((pad,), jnp.int32)])
  dst_p = jnp.concatenate([dst, jnp.full((pad,), 2 * n, jnp.int32)])
  srcs = src_p.reshape(NS, kc, C)
  dstp = []
  for p in range(NP):
    dloc = dst_p - p * half
    in_range = (dloc >= 0) & (dloc < half)
    dstp.append(jnp.where(in_range, dloc, half))
  dsts = jnp.stack(dstp).reshape(NP, NS, kc, C)

  x_split = jnp.transpose(x.reshape(n, 2, dh), (1, 0, 2))

  p1 = _seg_sum_split(x_split, srcs, dsts, half)
  h = _linear(p1, W1, b1, True, True, n)
  p2 = _seg_sum_split(h, srcs, dsts, half)
  return _linear(p2, W2, b2, False, False, n)


# speedup vs baseline: 2.1838x; 2.1838x over previous
import jax
import jax.numpy as jnp
from jax import lax
from jax.experimental import pallas as pl
from jax.experimental.pallas import tpu as pltpu
from jax.experimental.pallas import tpu_sc as plsc

NC = 2
NS = 16
C = 128


def _seg_sum_split(table, srcs, dsts, acc_rows):
  _, dh = table.shape
  _, _, kc, _ = srcs.shape
  assert kc % 2 == 0
  tot = 2 * acc_rows
  rpw = tot // NS
  ZC = 128
  zch = rpw // ZC

  mesh = plsc.VectorSubcoreMesh(core_axis_name="c", subcore_axis_name="s")

  def body(tbl_hbm, src_hbm, dst_hbm, zero_hbm, out_hbm,
           src_v, dst_v, bufs, zrows_v, acc, gsems, ssems):
    c = lax.axis_index("c")
    s = lax.axis_index("s")

    pltpu.sync_copy(zero_hbm, zrows_v)
    for z in range(zch):
      pltpu.sync_copy(zrows_v, acc.at[pl.ds((s * zch + z) * ZC, ZC)])
    plsc.subcore_barrier()

    pltpu.sync_copy(src_hbm.at[c, s], src_v)
    pltpu.sync_copy(dst_hbm.at[s], dst_v)

    def gather(j, b):
      pltpu.async_copy(tbl_hbm.at[src_v.at[j]], bufs[b], gsems[b])

    def gather_wait(j, b):
      pltpu.make_async_copy(tbl_hbm.at[src_v.at[j]], bufs[b], gsems[b]).wait()

    def scatter(j, b):
      pltpu.async_copy(bufs[b], acc.at[dst_v.at[j]], ssems[b], add=True)

    def scatter_wait(j, b):
      pltpu.make_async_copy(bufs[b], acc.at[dst_v.at[j]], ssems[b]).wait()

    gather(0, 0)

    def pair(i, carry):
      j = 2 * i
      @pl.when(i > 0)
      def _():
        scatter_wait(j - 1, 1)
      gather(j + 1, 1)
      gather_wait(j, 0)
      scatter(j, 0)
      scatter_wait(j, 0)
      @pl.when(i + 1 < kc // 2)
      def _():
        gather(j + 2, 0)
      gather_wait(j + 1, 1)
      scatter(j + 1, 1)
      return carry

    lax.fori_loop(0, kc // 2, pair, 0)
    scatter_wait(kc - 1, 1)
    plsc.subcore_barrier()

    pltpu.sync_copy(acc.at[pl.ds(s * rpw, rpw)],
                    out_hbm.at[c, pl.ds(s * rpw, rpw)])

  zeros = jnp.zeros((ZC, dh), jnp.bfloat16)
  return pl.kernel(
      body,
      out_type=jax.ShapeDtypeStruct((NC, tot, dh), jnp.bfloat16),
      mesh=mesh,
      compiler_params=pltpu.CompilerParams(use_tc_tiling_on_sc=False),
      scratch_types=[
          pltpu.VMEM((kc, C), jnp.int32),
          pltpu.VMEM((kc, C), jnp.int32),
          [pltpu.VMEM((C, dh), jnp.bfloat16)] * 2,
          pltpu.VMEM((ZC, dh), jnp.bfloat16),
          pltpu.VMEM_SHARED((tot, dh), jnp.bfloat16),
          [pltpu.SemaphoreType.DMA] * 2,
          [pltpu.SemaphoreType.DMA] * 2,
      ],
  )(table, srcs, dsts, zeros)


def _linear(p, w, b, relu, split_out, n):
  _, _, rows, dh = p.shape
  dout = w.shape[1]
  blk = 2000
  assert n % blk == 0

  def body(p_ref, w_ref, b_ref, o_ref):
    pf = p_ref[...].astype(jnp.float32)
    ssum = jnp.concatenate([pf[0, 0] + pf[0, 1], pf[1, 0] + pf[1, 1]], axis=1)
    y = lax.dot_general(ssum, w_ref[...], (((1,), (0,)), ((), ())),
                        preferred_element_type=jnp.float32,
                        precision=lax.Precision.HIGHEST)
    y = y + b_ref[...]
    if relu:
      y = jnp.maximum(y, 0.0)
    if split_out:
      yb = y.astype(jnp.bfloat16)
      o_ref[...] = yb.reshape(yb.shape[0], 2, dout // 2)
    else:
      o_ref[...] = y

  if split_out:
    out_shape = jax.ShapeDtypeStruct((n, 2, dout // 2), jnp.bfloat16)
    out_specs = pl.BlockSpec((blk, 2, dout // 2), lambda i: (i, 0, 0))
  else:
    out_shape = jax.ShapeDtypeStruct((n, dout), jnp.float32)
    out_specs = pl.BlockSpec((blk, dout), lambda i: (i, 0))

  return pl.pallas_call(
      body,
      grid=(n // blk,),
      in_specs=[
          pl.BlockSpec((2, 2, blk, dh), lambda i: (0, 0, i, 0)),
          pl.BlockSpec((dh * 2, dout), lambda i: (0, 0)),
          pl.BlockSpec((1, dout), lambda i: (0, 0)),
      ],
      out_specs=out_specs,
      out_shape=out_shape,
  )(p, w, b.reshape(1, dout))


def kernel(x, edge_index, W1, b1, W2, b2):
  n, d = x.shape
  dh = d // 2
  e = edge_index.shape[1]
  src = edge_index[0].astype(jnp.int32)
  dst = edge_index[1].astype(jnp.int32)

  kc = 2 * (-(-e // (NS * C * 2)))
  e_pad = kc * NS * C
  acc_rows = (n // 2048 + 1) * 2048

  pad = e_pad - e
  src_p = jnp.concatenate([src, jnp.zeros((pad,), jnp.int32)])
  dst_p = jnp.concatenate([dst, jnp.full((pad,), n, jnp.int32)])
  srcs = src_p.reshape(NS, kc, C)
  srcs2 = jnp.stack([2 * srcs, 2 * srcs + 1])
  parity = (jnp.arange(kc, dtype=jnp.int32) % 2)[None, :, None]
  dsts = dst_p.reshape(NS, kc, C) + parity * acc_rows

  tbl1 = x.astype(jnp.bfloat16).reshape(2 * n, dh)

  p1 = _seg_sum_split(tbl1, srcs2, dsts, acc_rows)
  h = _linear(p1.reshape(2, 2, acc_rows, dh), W1, b1, True, True, n)
  p2 = _seg_sum_split(h.reshape(2 * n, dh), srcs2, dsts, acc_rows)
  return _linear(p2.reshape(2, 2, acc_rows, dh), W2, b2, False, False, n)
